# Initial kernel scaffold; baseline (speedup 1.0000x reference)
import jax
import jax.numpy as jnp
from jax.experimental import pallas as pl


def kernel(x, W1, b1, W2, b2, Wd, bd):
    raise NotImplementedError("write your pallas kernel here")


# speedup vs baseline: 6.8431x; 6.8431x over previous
import functools

import jax
import jax.numpy as jnp
from jax import lax
from jax.experimental import pallas as pl
from jax.experimental.pallas import tpu as pltpu
from jax.experimental.pallas import tpu_sc as plsc

NPTS = 10000
NPAD = 10240
D = 128
K = 7
KP = 8
NEG = -1e30
RBLK = 256

_NC, _NS = 2, 16
NW = _NC * _NS
RPW = NPAD // NW
CH = 16
NCHUNK = RPW // CH


def _topk_body(featT_ref, fblk_ref, sqr_ref, sqc_ref, idx_ref):
    i = pl.program_id(0)
    feat = featT_ref[...].astype(jnp.bfloat16)
    fb = fblk_ref[...].astype(jnp.bfloat16)
    dot = lax.dot_general(fb, feat, (((1,), (1,)), ((), ())),
                          preferred_element_type=jnp.float32)
    d = (sqc_ref[...] + sqr_ref[...]) - 2.0 * dot
    col = lax.broadcasted_iota(jnp.int32, (RBLK, NPAD), 1)
    row = lax.broadcasted_iota(jnp.int32, (RBLK, NPAD), 0) + i * RBLK
    d = jnp.where((col == row) | (col >= NPTS), jnp.inf, d)

    picks = []
    for _k in range(K):
        m = jnp.min(d, axis=1, keepdims=True)
        cand = jnp.where(d == m, col, NPAD)
        a = jnp.min(cand, axis=1, keepdims=True)
        picks.append(a)
        d = jnp.where(col == a, jnp.inf, d)
    picks.append(picks[-1])
    idx_ref[...] = jnp.concatenate(picks, axis=1)


def _topk(feat, sq):
    return pl.pallas_call(
        _topk_body,
        grid=(NPAD // RBLK,),
        in_specs=[
            pl.BlockSpec((NPAD, D), lambda i: (0, 0)),
            pl.BlockSpec((RBLK, D), lambda i: (i, 0)),
            pl.BlockSpec((1, NPAD), lambda i: (0, 0)),
            pl.BlockSpec((RBLK, 1), lambda i: (i, 0)),
        ],
        out_specs=pl.BlockSpec((RBLK, KP), lambda i: (i, 0)),
        out_shape=jax.ShapeDtypeStruct((NPAD, KP), jnp.int32),
    )(feat, feat, sq.reshape(1, NPAD), sq.reshape(NPAD, 1))


def _sq_body(f_ref, sq_ref):
    f = f_ref[...]
    sq_ref[...] = jnp.sum(f * f, axis=1, keepdims=True)


def _sq(feat):
    blk = min(2048, NPAD)
    return pl.pallas_call(
        _sq_body,
        grid=(NPAD // blk,),
        in_specs=[pl.BlockSpec((blk, D), lambda i: (i, 0))],
        out_specs=pl.BlockSpec((blk, 1), lambda i: (i, 0)),
        out_shape=jax.ShapeDtypeStruct((NPAD, 1), jnp.float32),
    )(feat)


def _be_body(idx_hbm, f_hbm, e_hbm, idx_v, xj_v, xi_v, e_v, sem):
    cid = lax.axis_index("c")
    sid = lax.axis_index("s")
    wid = sid * _NC + cid
    for ch in range(NCHUNK):
        base = wid * RPW + ch * CH
        pltpu.sync_copy(idx_hbm.at[pl.ds(base * KP, CH * KP)], idx_v)
        pltpu.async_copy(f_hbm.at[idx_v], xj_v, sem).wait()
        pltpu.sync_copy(f_hbm.at[pl.ds(base, CH)], xi_v)

        def body(r, carry):
            for f in range(D // 16):
                sl = pl.ds(f * 16, 16)
                sl2 = pl.ds(D + f * 16, 16)
                xi = xi_v[r, sl]
                for nb in range(KP):
                    xj = xj_v[r * KP + nb, sl]
                    e_v[r * KP + nb, sl] = xi
                    e_v[r * KP + nb, sl2] = xj - xi
            return carry

        lax.fori_loop(0, CH, body, 0)
        pltpu.sync_copy(e_v, e_hbm.at[pl.ds(base * KP, CH * KP)])


def _build_e(idx_flat, feat):
    mesh = plsc.VectorSubcoreMesh(core_axis_name="c", subcore_axis_name="s")
    return pl.kernel(
        _be_body,
        out_type=jax.ShapeDtypeStruct((NPAD * KP, 2 * D), jnp.float32),
        mesh=mesh,
        scratch_types=[
            pltpu.VMEM((CH * KP,), jnp.int32),
            pltpu.VMEM((CH * KP, D), jnp.float32),
            pltpu.VMEM((CH, D), jnp.float32),
            pltpu.VMEM((CH * KP, 2 * D), jnp.float32),
            pltpu.SemaphoreType.DMA,
        ],
    )(idx_flat, feat)


_ECB = 128


def _ec_body(e_ref, w_ref, b_ref, h_ref, sq_ref):
    t = (jnp.dot(e_ref[...], w_ref[...],
                 precision=jax.lax.Precision.HIGHEST,
                 preferred_element_type=jnp.float32) + b_ref[...])
    hm = jnp.max(t.reshape(_ECB, KP, D), axis=1)
    h = jnp.maximum(hm, 0.0)
    h_ref[...] = h
    sq_ref[...] = jnp.sum(h * h, axis=1, keepdims=True)


def _ec(e, W, b):
    return pl.pallas_call(
        _ec_body,
        grid=(NPAD // _ECB,),
        in_specs=[
            pl.BlockSpec((_ECB * KP, 2 * D), lambda i: (i, 0)),
            pl.BlockSpec((2 * D, D), lambda i: (0, 0)),
            pl.BlockSpec((1, D), lambda i: (0, 0)),
        ],
        out_specs=[
            pl.BlockSpec((_ECB, D), lambda i: (i, 0)),
            pl.BlockSpec((_ECB, 1), lambda i: (i, 0)),
        ],
        out_shape=[
            jax.ShapeDtypeStruct((NPAD, D), jnp.float32),
            jax.ShapeDtypeStruct((NPAD, 1), jnp.float32),
        ],
    )(e, W, b.reshape(1, D))


def _dec_body(h_ref, wd_ref, bd_ref, o_ref):
    o_ref[...] = (jnp.dot(h_ref[...], wd_ref[...],
                          precision=jax.lax.Precision.HIGHEST,
                          preferred_element_type=jnp.float32) + bd_ref[...])


def _dec(h, Wd, bd):
    blk = min(2048, NPAD)
    return pl.pallas_call(
        _dec_body,
        grid=(NPAD // blk,),
        in_specs=[
            pl.BlockSpec((blk, D), lambda i: (i, 0)),
            pl.BlockSpec((D, 1), lambda i: (0, 0)),
            pl.BlockSpec((1, 1), lambda i: (0, 0)),
        ],
        out_specs=pl.BlockSpec((blk, 1), lambda i: (i, 0)),
        out_shape=jax.ShapeDtypeStruct((NPAD, 1), jnp.float32),
    )(h, Wd, bd.reshape(1, 1))


def _layer(feat, sq, W, b):
    idx = _topk(feat, sq)
    e = _build_e(idx.reshape(-1), feat)
    return _ec(e, W, b)


def kernel(x, W1, b1, W2, b2, Wd, bd):
    xp = jnp.pad(x, ((0, NPAD - NPTS), (0, 0)))
    sq1 = _sq(xp)
    h1, sq2 = _layer(xp, sq1, W1, b1)
    h2, _ = _layer(h1, sq2, W2, b2)
    out = _dec(h2, Wd, bd)
    return out[:NPTS]
